# Initial kernel scaffold; baseline (speedup 1.0000x reference)
import jax
import jax.numpy as jnp
from jax.experimental import pallas as pl


def kernel(h, x, edges, vel, edge_attr_orig, charges, num_nodes, emb, Ws1, bs1, Ws2, bs2, Wf1, bf1, Wg, bg, Wb, bb, Wf2, bf2, We, be, Wm, bm, Wn, bn, Wo, bo):
    raise NotImplementedError("write your pallas kernel here")


# speedup vs baseline: 3.3641x; 3.3641x over previous
import functools

import jax
import jax.numpy as jnp
from jax import lax
from jax.experimental import pallas as pl
from jax.experimental.pallas import tpu as pltpu
from jax.experimental.pallas import tpu_sc as plsc

N = 50000
E = 800000
D = 3
H = 64
GRAPH = 100
NGB = 20
NB = NGB * GRAPH
NBLK = N // NB
EB = 8000
EBLK = E // EB

NC = 2
NS = 16
NW = NC * NS
EPW = E // NW
GCH = 128
GFULL = EPW // GCH
GTAIL = EPW - GFULL * GCH
EPT = E // NS
SFULL = EPT // GCH
STAIL = EPT - SFULL * GCH
NPT = N // NS

TW = 80


def _node_kernel(x_ref, vel_ref, ch_ref, emb_ref, Ws1, bs1, Ws2, bs2,
                 Wf1, bf1, Wg, bg, Wb, bb, Wf2, bf2, We, be,
                 Wms, Wmr, bm, stab_ref, rtab_ref, hn_ref, r9_ref):
    f32 = jnp.float32
    x = x_ref[...]
    vel = vel_ref[...]
    inputs = jnp.concatenate([x, vel], axis=-1)
    hs = jnp.tanh(jnp.dot(inputs, Ws1[...], preferred_element_type=f32) + bs1[...])
    hs = jnp.dot(hs, Ws2[...], preferred_element_type=f32) + bs2[...]
    row_g = lax.broadcasted_iota(jnp.int32, (NGB, NB), 0)
    col_g = lax.broadcasted_iota(jnp.int32, (NGB, NB), 1) // GRAPH
    P = jnp.where(row_g == col_g, 1.0 / GRAPH, 0.0).astype(f32)
    gsum = jnp.dot(P, hs, preferred_element_type=f32)
    row_n = lax.broadcasted_iota(jnp.int32, (NB, NGB), 0) // GRAPH
    col_n = lax.broadcasted_iota(jnp.int32, (NB, NGB), 1)
    Q = jnp.where(row_n == col_n, 1.0, 0.0).astype(f32)
    gsum_rep = jnp.dot(Q, gsum, preferred_element_type=f32)
    idx = ch_ref[...] + 1
    emb_class = ((idx == 0).astype(f32) * emb_ref[0:1, :]
                 + (idx == 1).astype(f32) * emb_ref[1:2, :]
                 + (idx == 2).astype(f32) * emb_ref[2:3, :])
    fi = jnp.concatenate([x, vel, emb_class], axis=-1)
    h1 = jax.nn.relu(jnp.dot(fi, Wf1[...], preferred_element_type=f32) + bf1[...])
    gamma = jnp.dot(gsum_rep, Wg[...], preferred_element_type=f32) + bg[...]
    beta = jnp.dot(gsum_rep, Wb[...], preferred_element_type=f32) + bb[...]
    h1 = gamma * h1 + beta
    field = jnp.dot(h1, Wf2[...], preferred_element_type=f32) + bf2[...]
    eps = 1e-6
    a = vel / (jnp.sqrt(jnp.sum(vel * vel, axis=-1, keepdims=True)) + eps)
    b = field - jnp.sum(field * a, axis=-1, keepdims=True) * a
    b = b / (jnp.sqrt(jnp.sum(b * b, axis=-1, keepdims=True)) + eps)
    c = jnp.concatenate([
        a[:, 1:2] * b[:, 2:3] - a[:, 2:3] * b[:, 1:2],
        a[:, 2:3] * b[:, 0:1] - a[:, 0:1] * b[:, 2:3],
        a[:, 0:1] * b[:, 1:2] - a[:, 1:2] * b[:, 0:1],
    ], axis=-1)
    r9 = jnp.concatenate([a, b, c], axis=-1)

    def dot3(u, v):
        return jnp.sum(u * v, axis=-1, keepdims=True)

    rel_feat = jnp.concatenate([
        dot3(a, x), dot3(b, x), dot3(c, x),
        dot3(a, vel), dot3(b, vel), dot3(c, vel),
        dot3(a, field), dot3(b, field), dot3(c, field),
    ], axis=-1)
    hn = jax.nn.relu(jnp.dot(rel_feat, We[...], preferred_element_type=f32) + be[...])
    A = jnp.dot(hn, Wms[...], preferred_element_type=f32)
    B = jnp.dot(hn, Wmr[...], preferred_element_type=f32) + bm[...]
    zpad = jnp.zeros((NB, TW - 67), dtype=f32)
    stab_ref[...] = jnp.concatenate([A, x, zpad], axis=-1)
    rtab_ref[...] = jnp.concatenate([B, x, r9, jnp.zeros((NB, TW - 76), f32)], axis=-1)
    hn_ref[...] = hn
    r9_ref[...] = r9


def _edge_kernel(gs_ref, gr_ref, ea_ref, Wme, m_ref):
    f32 = jnp.float32
    gs = gs_ref[...]
    gr = gr_ref[...]
    A = gs[:, 0:64]
    xs = gs[:, 64:67]
    B = gr[:, 0:64]
    xr = gr[:, 64:67]
    ar = gr[:, 67:70]
    br = gr[:, 70:73]
    cr = gr[:, 73:76]
    rp = xs - xr
    rl = jnp.concatenate([
        jnp.sum(ar * rp, axis=-1, keepdims=True),
        jnp.sum(br * rp, axis=-1, keepdims=True),
        jnp.sum(cr * rp, axis=-1, keepdims=True),
    ], axis=-1)
    dist = jnp.sqrt(jnp.sum(rp * rp, axis=-1, keepdims=True))
    eattr = jnp.concatenate([rl, dist, ea_ref[...]], axis=-1)
    m = jax.nn.relu(A + B + jnp.dot(eattr, Wme[...], preferred_element_type=f32))
    m_ref[0] = m[:, 0:32]
    m_ref[1] = m[:, 32:64]


def _final_kernel(hn_ref, agg_ref, r9_ref, x_ref, Wn, bn, Wo, bo, out_ref):
    f32 = jnp.float32
    hn = hn_ref[...]
    agg = jnp.concatenate([agg_ref[0], agg_ref[1]], axis=-1)
    cat = jnp.concatenate([hn, agg], axis=-1)
    hn2 = hn + jax.nn.relu(jnp.dot(cat, Wn[...], preferred_element_type=f32) + bn[...])
    pred = jnp.dot(hn2, Wo[...], preferred_element_type=f32) + bo[...]
    a = r9_ref[:, 0:3]
    b = r9_ref[:, 3:6]
    c = r9_ref[:, 6:9]
    out_ref[...] = (x_ref[...] + pred[:, 0:1] * a + pred[:, 1:2] * b
                    + pred[:, 2:3] * c)


def _sc_gather(sidx, ridx, stab, rtab, gs_out, gr_out, idx_v, rows_v,
               idx_t, rows_t, sem):
    wid = lax.axis_index("s") * NC + lax.axis_index("c")
    base = wid * EPW

    def chunk(cb, n, iv, rv):
        pltpu.sync_copy(sidx.at[pl.ds(cb, n)], iv)
        pltpu.async_copy(stab.at[iv], rv, sem).wait()
        pltpu.sync_copy(rv, gs_out.at[pl.ds(cb, n)])
        pltpu.sync_copy(ridx.at[pl.ds(cb, n)], iv)
        pltpu.async_copy(rtab.at[iv], rv, sem).wait()
        pltpu.sync_copy(rv, gr_out.at[pl.ds(cb, n)])

    def body(k, _):
        chunk(base + k * GCH, GCH, idx_v, rows_v)
        return 0

    lax.fori_loop(0, GFULL, body, 0)
    chunk(base + GFULL * GCH, GTAIL, idx_t, rows_t)


def _sc_scatter(ridx, m_hbm, zeros_hbm, agg_out, acc, idx_v, rows_v,
                idx_t, rows_t, sem):
    c = lax.axis_index("c")
    s = lax.axis_index("s")
    rbase = s * NPT
    pltpu.sync_copy(zeros_hbm.at[pl.ds(rbase, NPT)], acc.at[pl.ds(rbase, NPT)])
    plsc.subcore_barrier()
    base = s * EPT

    def chunk(cb, iv, rv):
        pltpu.sync_copy(ridx.at[cb], iv)
        pltpu.sync_copy(m_hbm.at[c, cb], rv)
        pltpu.sync_copy(rv, acc.at[iv], add=True)

    def body(k, _):
        cb = base + k * GCH
        chunk(pl.ds(cb, GCH), idx_v, rows_v)
        return 0

    lax.fori_loop(0, SFULL, body, 0)
    chunk(pl.ds(base + SFULL * GCH, STAIL), idx_t, rows_t)
    plsc.subcore_barrier()
    pltpu.sync_copy(acc.at[pl.ds(rbase, NPT)], agg_out.at[c, pl.ds(rbase, NPT)])


def kernel(h, x, edges, vel, edge_attr_orig, charges, num_nodes, emb, Ws1,
           bs1, Ws2, bs2, Wf1, bf1, Wg, bg, Wb, bb, Wf2, bf2, We, be, Wm, bm,
           Wn, bn, Wo, bo):
    f32 = jnp.float32
    row = lambda v: v.reshape(1, -1).astype(f32)
    Wms, Wmr, Wme = Wm[0:64], Wm[64:128], Wm[128:136]

    grid1 = pl.pallas_call(
        _node_kernel,
        grid=(NBLK,),
        in_specs=[
            pl.BlockSpec((NB, D), lambda i: (i, 0)),
            pl.BlockSpec((NB, D), lambda i: (i, 0)),
            pl.BlockSpec((NB, 1), lambda i: (i, 0)),
        ] + [pl.BlockSpec(w.shape, lambda i: tuple(0 for _ in w.shape))
             for w in (emb, Ws1, row(bs1), Ws2, row(bs2), Wf1, row(bf1), Wg,
                       row(bg), Wb, row(bb), Wf2, row(bf2), We, row(be), Wms,
                       Wmr, row(bm))],
        out_specs=[
            pl.BlockSpec((NB, TW), lambda i: (i, 0)),
            pl.BlockSpec((NB, TW), lambda i: (i, 0)),
            pl.BlockSpec((NB, H), lambda i: (i, 0)),
            pl.BlockSpec((NB, 9), lambda i: (i, 0)),
        ],
        out_shape=[
            jax.ShapeDtypeStruct((N, TW), f32),
            jax.ShapeDtypeStruct((N, TW), f32),
            jax.ShapeDtypeStruct((N, H), f32),
            jax.ShapeDtypeStruct((N, 9), f32),
        ],
    )
    stab, rtab, hn, r9 = grid1(
        x.astype(f32), vel.astype(f32), charges.astype(jnp.int32), emb, Ws1,
        row(bs1), Ws2, row(bs2), Wf1, row(bf1), Wg, row(bg), Wb, row(bb), Wf2,
        row(bf2), We, row(be), Wms, Wmr, row(bm))

    sidx = edges[0]
    ridx = edges[1]

    gather = functools.partial(
        pl.kernel,
        mesh=plsc.VectorSubcoreMesh(core_axis_name="c", subcore_axis_name="s"),
        compiler_params=pltpu.CompilerParams(use_tc_tiling_on_sc=False),
        out_type=[
            jax.ShapeDtypeStruct((E, TW), f32),
            jax.ShapeDtypeStruct((E, TW), f32),
        ],
        scratch_types=[
            pltpu.VMEM((GCH,), jnp.int32),
            pltpu.VMEM((GCH, TW), f32),
            pltpu.VMEM((GTAIL,), jnp.int32),
            pltpu.VMEM((GTAIL, TW), f32),
            pltpu.SemaphoreType.DMA,
        ],
    )(_sc_gather)
    gs, gr = gather(sidx, ridx, stab, rtab)

    grid3 = pl.pallas_call(
        _edge_kernel,
        grid=(EBLK,),
        in_specs=[
            pl.BlockSpec((EB, TW), lambda i: (i, 0)),
            pl.BlockSpec((EB, TW), lambda i: (i, 0)),
            pl.BlockSpec((EB, 4), lambda i: (i, 0)),
            pl.BlockSpec((8, H), lambda i: (0, 0)),
        ],
        out_specs=pl.BlockSpec((2, EB, 32), lambda i: (0, i, 0)),
        out_shape=jax.ShapeDtypeStruct((2, E, 32), f32),
    )
    m2 = grid3(gs, gr, edge_attr_orig.astype(f32), Wme)

    scatter = functools.partial(
        pl.kernel,
        mesh=plsc.VectorSubcoreMesh(core_axis_name="c", subcore_axis_name="s"),
        compiler_params=pltpu.CompilerParams(use_tc_tiling_on_sc=False),
        out_type=jax.ShapeDtypeStruct((2, N, 32), f32),
        scratch_types=[
            pltpu.VMEM_SHARED((N, 32), f32),
            pltpu.VMEM((GCH,), jnp.int32),
            pltpu.VMEM((GCH, 32), f32),
            pltpu.VMEM((STAIL,), jnp.int32),
            pltpu.VMEM((STAIL, 32), f32),
            pltpu.SemaphoreType.DMA,
        ],
    )(_sc_scatter)
    agg2 = scatter(ridx, m2, jnp.zeros((N, 32), f32))

    grid5 = pl.pallas_call(
        _final_kernel,
        grid=(NBLK,),
        in_specs=[
            pl.BlockSpec((NB, H), lambda i: (i, 0)),
            pl.BlockSpec((2, NB, 32), lambda i: (0, i, 0)),
            pl.BlockSpec((NB, 9), lambda i: (i, 0)),
            pl.BlockSpec((NB, D), lambda i: (i, 0)),
            pl.BlockSpec((2 * H, H), lambda i: (0, 0)),
            pl.BlockSpec((1, H), lambda i: (0, 0)),
            pl.BlockSpec((H, D), lambda i: (0, 0)),
            pl.BlockSpec((1, D), lambda i: (0, 0)),
        ],
        out_specs=pl.BlockSpec((NB, D), lambda i: (i, 0)),
        out_shape=jax.ShapeDtypeStruct((N, D), f32),
    )
    return grid5(hn, agg2, r9, x.astype(f32), Wn, row(bn), Wo, row(bo))


# speedup vs baseline: 3.8551x; 1.1460x over previous
import functools

import jax
import jax.numpy as jnp
from jax import lax
from jax.experimental import pallas as pl
from jax.experimental.pallas import tpu as pltpu
from jax.experimental.pallas import tpu_sc as plsc

N = 50000
E = 800000
D = 3
H = 64
GRAPH = 100
NGB = 20
NB = NGB * GRAPH
NBLK = N // NB
EB = 8000
EBLK = E // EB

NC = 2
NS = 16
NW = NC * NS
GCH = 128
NCHUNK = E // GCH
GFULL = NCHUNK // NW
GREM = NCHUNK - GFULL * NW
SFULL = NCHUNK // NS
SREM = NCHUNK - SFULL * NS
EPT = E // NS
SFULL2 = EPT // GCH
STAIL = EPT - SFULL2 * GCH
NPT = N // NS

TW = 128


def _node_kernel(x_ref, vel_ref, ch_ref, emb_ref, Ws1, bs1, Ws2, bs2,
                 Wf1, bf1, Wg, bg, Wb, bb, Wf2, bf2, We, be,
                 Wms, Wmr, bm, stab_ref, rtab_ref, hn_ref, r9_ref):
    f32 = jnp.float32
    x = x_ref[...]
    vel = vel_ref[...]
    inputs = jnp.concatenate([x, vel], axis=-1)
    hs = jnp.tanh(jnp.dot(inputs, Ws1[...], preferred_element_type=f32) + bs1[...])
    hs = jnp.dot(hs, Ws2[...], preferred_element_type=f32) + bs2[...]
    row_g = lax.broadcasted_iota(jnp.int32, (NGB, NB), 0)
    col_g = lax.broadcasted_iota(jnp.int32, (NGB, NB), 1) // GRAPH
    P = jnp.where(row_g == col_g, 1.0 / GRAPH, 0.0).astype(f32)
    gsum = jnp.dot(P, hs, preferred_element_type=f32)
    row_n = lax.broadcasted_iota(jnp.int32, (NB, NGB), 0) // GRAPH
    col_n = lax.broadcasted_iota(jnp.int32, (NB, NGB), 1)
    Q = jnp.where(row_n == col_n, 1.0, 0.0).astype(f32)
    gsum_rep = jnp.dot(Q, gsum, preferred_element_type=f32)
    idx = ch_ref[...] + 1
    emb_class = ((idx == 0).astype(f32) * emb_ref[0:1, :]
                 + (idx == 1).astype(f32) * emb_ref[1:2, :]
                 + (idx == 2).astype(f32) * emb_ref[2:3, :])
    fi = jnp.concatenate([x, vel, emb_class], axis=-1)
    h1 = jax.nn.relu(jnp.dot(fi, Wf1[...], preferred_element_type=f32) + bf1[...])
    gamma = jnp.dot(gsum_rep, Wg[...], preferred_element_type=f32) + bg[...]
    beta = jnp.dot(gsum_rep, Wb[...], preferred_element_type=f32) + bb[...]
    h1 = gamma * h1 + beta
    field = jnp.dot(h1, Wf2[...], preferred_element_type=f32) + bf2[...]
    eps = 1e-6
    a = vel / (jnp.sqrt(jnp.sum(vel * vel, axis=-1, keepdims=True)) + eps)
    b = field - jnp.sum(field * a, axis=-1, keepdims=True) * a
    b = b / (jnp.sqrt(jnp.sum(b * b, axis=-1, keepdims=True)) + eps)
    c = jnp.concatenate([
        a[:, 1:2] * b[:, 2:3] - a[:, 2:3] * b[:, 1:2],
        a[:, 2:3] * b[:, 0:1] - a[:, 0:1] * b[:, 2:3],
        a[:, 0:1] * b[:, 1:2] - a[:, 1:2] * b[:, 0:1],
    ], axis=-1)
    r9 = jnp.concatenate([a, b, c], axis=-1)

    def dot3(u, v):
        return jnp.sum(u * v, axis=-1, keepdims=True)

    rel_feat = jnp.concatenate([
        dot3(a, x), dot3(b, x), dot3(c, x),
        dot3(a, vel), dot3(b, vel), dot3(c, vel),
        dot3(a, field), dot3(b, field), dot3(c, field),
    ], axis=-1)
    hn = jax.nn.relu(jnp.dot(rel_feat, We[...], preferred_element_type=f32) + be[...])
    A = jnp.dot(hn, Wms[...], preferred_element_type=f32)
    B = jnp.dot(hn, Wmr[...], preferred_element_type=f32) + bm[...]
    zpad = jnp.zeros((NB, TW - 67), dtype=f32)
    stab_ref[...] = jnp.concatenate([A, x, zpad], axis=-1)
    rtab_ref[...] = jnp.concatenate([B, x, r9, jnp.zeros((NB, TW - 76), f32)], axis=-1)
    hn_ref[...] = hn
    r9_ref[...] = r9


def _edge_kernel(gs_ref, gr_ref, ea_ref, Wme, m_ref):
    f32 = jnp.float32
    gs = gs_ref[...]
    gr = gr_ref[...]
    A = gs[:, 0:64]
    xs = gs[:, 64:67]
    B = gr[:, 0:64]
    xr = gr[:, 64:67]
    ar = gr[:, 67:70]
    br = gr[:, 70:73]
    cr = gr[:, 73:76]
    rp = xs - xr
    rl = jnp.concatenate([
        jnp.sum(ar * rp, axis=-1, keepdims=True),
        jnp.sum(br * rp, axis=-1, keepdims=True),
        jnp.sum(cr * rp, axis=-1, keepdims=True),
    ], axis=-1)
    dist = jnp.sqrt(jnp.sum(rp * rp, axis=-1, keepdims=True))
    eattr = jnp.concatenate([rl, dist, ea_ref[...]], axis=-1)
    m = jax.nn.relu(A + B + jnp.dot(eattr, Wme[...], preferred_element_type=f32))
    m_ref[0] = m[:, 0:32]
    m_ref[1] = m[:, 32:64]


def _final_kernel(hn_ref, agg_ref, r9_ref, x_ref, Wn, bn, Wo, bo, out_ref):
    f32 = jnp.float32
    hn = hn_ref[...]
    cat = jnp.concatenate([hn, agg_ref[0], agg_ref[1]], axis=-1)
    hn2 = hn + jax.nn.relu(jnp.dot(cat, Wn[...], preferred_element_type=f32) + bn[...])
    pred = jnp.dot(hn2, Wo[...], preferred_element_type=f32) + bo[...]
    a = r9_ref[:, 0:3]
    b = r9_ref[:, 3:6]
    c = r9_ref[:, 6:9]
    out_ref[...] = (x_ref[...] + pred[:, 0:1] * a + pred[:, 1:2] * b
                    + pred[:, 2:3] * c)


def _sc_gather(sidx, ridx, stab, rtab, gs_out, gr_out, idx_v, rows_v, sem):
    wid = lax.axis_index("s") * NC + lax.axis_index("c")

    def chunk(j):
        cb = j * GCH
        pltpu.sync_copy(sidx.at[pl.ds(cb, GCH)], idx_v)
        pltpu.async_copy(stab.at[idx_v], rows_v, sem).wait()
        pltpu.sync_copy(rows_v, gs_out.at[pl.ds(cb, GCH)])
        pltpu.sync_copy(ridx.at[pl.ds(cb, GCH)], idx_v)
        pltpu.async_copy(rtab.at[idx_v], rows_v, sem).wait()
        pltpu.sync_copy(rows_v, gr_out.at[pl.ds(cb, GCH)])

    def body(k, _):
        chunk(wid + k * NW)
        return 0

    lax.fori_loop(0, GFULL, body, 0)
    pl.when(wid < GREM)(lambda: chunk(GFULL * NW + wid))


def _sc_scatter(ridx, m_hbm, zeros_hbm, agg_out, acc, idx_v, rows_v,
                idx_t, rows_t, sem):
    c = lax.axis_index("c")
    s = lax.axis_index("s")
    rbase = s * NPT
    pltpu.sync_copy(zeros_hbm.at[pl.ds(rbase, NPT)], acc.at[pl.ds(rbase, NPT)])
    plsc.subcore_barrier()
    base = s * EPT

    def chunk(cb, iv, rv):
        pltpu.sync_copy(ridx.at[cb], iv)
        pltpu.sync_copy(m_hbm.at[c, cb], rv)
        pltpu.sync_copy(rv, acc.at[iv], add=True)

    def body(k, _):
        cb = base + k * GCH
        chunk(pl.ds(cb, GCH), idx_v, rows_v)
        return 0

    lax.fori_loop(0, SFULL, body, 0)
    chunk(pl.ds(base + SFULL * GCH, STAIL), idx_t, rows_t)
    plsc.subcore_barrier()
    pltpu.sync_copy(acc.at[pl.ds(rbase, NPT)], agg_out.at[c, pl.ds(rbase, NPT)])


def kernel(h, x, edges, vel, edge_attr_orig, charges, num_nodes, emb, Ws1,
           bs1, Ws2, bs2, Wf1, bf1, Wg, bg, Wb, bb, Wf2, bf2, We, be, Wm, bm,
           Wn, bn, Wo, bo):
    f32 = jnp.float32
    row = lambda v: v.reshape(1, -1).astype(f32)
    Wms, Wmr, Wme = Wm[0:64], Wm[64:128], Wm[128:136]

    grid1 = pl.pallas_call(
        _node_kernel,
        grid=(NBLK,),
        in_specs=[
            pl.BlockSpec((NB, D), lambda i: (i, 0)),
            pl.BlockSpec((NB, D), lambda i: (i, 0)),
            pl.BlockSpec((NB, 1), lambda i: (i, 0)),
        ] + [pl.BlockSpec(w.shape, lambda i: tuple(0 for _ in w.shape))
             for w in (emb, Ws1, row(bs1), Ws2, row(bs2), Wf1, row(bf1), Wg,
                       row(bg), Wb, row(bb), Wf2, row(bf2), We, row(be), Wms,
                       Wmr, row(bm))],
        out_specs=[
            pl.BlockSpec((NB, TW), lambda i: (i, 0)),
            pl.BlockSpec((NB, TW), lambda i: (i, 0)),
            pl.BlockSpec((NB, H), lambda i: (i, 0)),
            pl.BlockSpec((NB, 9), lambda i: (i, 0)),
        ],
        out_shape=[
            jax.ShapeDtypeStruct((N, TW), f32),
            jax.ShapeDtypeStruct((N, TW), f32),
            jax.ShapeDtypeStruct((N, H), f32),
            jax.ShapeDtypeStruct((N, 9), f32),
        ],
    )
    stab, rtab, hn, r9 = grid1(
        x.astype(f32), vel.astype(f32), charges.astype(jnp.int32), emb, Ws1,
        row(bs1), Ws2, row(bs2), Wf1, row(bf1), Wg, row(bg), Wb, row(bb), Wf2,
        row(bf2), We, row(be), Wms, Wmr, row(bm))

    sidx = edges[0]
    ridx = edges[1]

    gather = functools.partial(
        pl.kernel,
        mesh=plsc.VectorSubcoreMesh(core_axis_name="c", subcore_axis_name="s"),
        out_type=[
            jax.ShapeDtypeStruct((E, TW), f32),
            jax.ShapeDtypeStruct((E, TW), f32),
        ],
        scratch_types=[
            pltpu.VMEM((GCH,), jnp.int32),
            pltpu.VMEM((GCH, TW), f32),
            pltpu.SemaphoreType.DMA,
        ],
    )(_sc_gather)
    gs, gr = gather(sidx, ridx, stab, rtab)

    grid3 = pl.pallas_call(
        _edge_kernel,
        grid=(EBLK,),
        in_specs=[
            pl.BlockSpec((EB, TW), lambda i: (i, 0)),
            pl.BlockSpec((EB, TW), lambda i: (i, 0)),
            pl.BlockSpec((EB, 4), lambda i: (i, 0)),
            pl.BlockSpec((8, H), lambda i: (0, 0)),
        ],
        out_specs=pl.BlockSpec((2, EB, 32), lambda i: (0, i, 0)),
        out_shape=jax.ShapeDtypeStruct((2, E, 32), f32),
    )
    m2 = grid3(gs, gr, edge_attr_orig.astype(f32), Wme)

    scatter = functools.partial(
        pl.kernel,
        mesh=plsc.VectorSubcoreMesh(core_axis_name="c", subcore_axis_name="s"),
        compiler_params=pltpu.CompilerParams(use_tc_tiling_on_sc=False),
        out_type=jax.ShapeDtypeStruct((2, N, 32), f32),
        scratch_types=[
            pltpu.VMEM_SHARED((N, 32), f32),
            pltpu.VMEM((GCH,), jnp.int32),
            pltpu.VMEM((GCH, 32), f32),
            pltpu.VMEM((STAIL,), jnp.int32),
            pltpu.VMEM((STAIL, 32), f32),
            pltpu.SemaphoreType.DMA,
        ],
    )(_sc_scatter)
    agg2 = scatter(ridx, m2, jnp.zeros((N, 32), f32))

    grid5 = pl.pallas_call(
        _final_kernel,
        grid=(NBLK,),
        in_specs=[
            pl.BlockSpec((NB, H), lambda i: (i, 0)),
            pl.BlockSpec((2, NB, 32), lambda i: (0, i, 0)),
            pl.BlockSpec((NB, 9), lambda i: (i, 0)),
            pl.BlockSpec((NB, D), lambda i: (i, 0)),
            pl.BlockSpec((2 * H, H), lambda i: (0, 0)),
            pl.BlockSpec((1, H), lambda i: (0, 0)),
            pl.BlockSpec((H, D), lambda i: (0, 0)),
            pl.BlockSpec((1, D), lambda i: (0, 0)),
        ],
        out_specs=pl.BlockSpec((NB, D), lambda i: (i, 0)),
        out_shape=jax.ShapeDtypeStruct((N, D), f32),
    )
    return grid5(hn, agg2, r9, x.astype(f32), Wn, row(bn), Wo, row(bo))


# speedup vs baseline: 4.7343x; 1.2281x over previous
import functools

import jax
import jax.numpy as jnp
from jax import lax
from jax.experimental import pallas as pl
from jax.experimental.pallas import tpu as pltpu
from jax.experimental.pallas import tpu_sc as plsc

N = 50000
E = 800000
D = 3
H = 64
GRAPH = 100
NGB = 20
NB = NGB * GRAPH
NBLK = N // NB
EB = 8000
EBLK = E // EB

NC = 2
NS = 16
NW = NC * NS
GCH = 128
NCHUNK = E // GCH
GFULL = NCHUNK // NW
GREM = NCHUNK - GFULL * NW
SFULL = NCHUNK // NS
SREM = NCHUNK - SFULL * NS
EPT = E // NS
SFULL2 = EPT // GCH
STAIL = EPT - SFULL2 * GCH
NPT = N // NS

TW = 128


def _node_kernel(x_ref, vel_ref, ch_ref, emb_ref, Ws1, bs1, Ws2, bs2,
                 Wf1, bf1, Wg, bg, Wb, bb, Wf2, bf2, We, be,
                 Wms, Wmr, bm, stab_ref, rtab_ref, hn_ref, r9_ref):
    f32 = jnp.float32
    x = x_ref[...]
    vel = vel_ref[...]
    inputs = jnp.concatenate([x, vel], axis=-1)
    hs = jnp.tanh(jnp.dot(inputs, Ws1[...], preferred_element_type=f32) + bs1[...])
    hs = jnp.dot(hs, Ws2[...], preferred_element_type=f32) + bs2[...]
    row_g = lax.broadcasted_iota(jnp.int32, (NGB, NB), 0)
    col_g = lax.broadcasted_iota(jnp.int32, (NGB, NB), 1) // GRAPH
    P = jnp.where(row_g == col_g, 1.0 / GRAPH, 0.0).astype(f32)
    gsum = jnp.dot(P, hs, preferred_element_type=f32)
    row_n = lax.broadcasted_iota(jnp.int32, (NB, NGB), 0) // GRAPH
    col_n = lax.broadcasted_iota(jnp.int32, (NB, NGB), 1)
    Q = jnp.where(row_n == col_n, 1.0, 0.0).astype(f32)
    gsum_rep = jnp.dot(Q, gsum, preferred_element_type=f32)
    idx = ch_ref[...] + 1
    emb_class = ((idx == 0).astype(f32) * emb_ref[0:1, :]
                 + (idx == 1).astype(f32) * emb_ref[1:2, :]
                 + (idx == 2).astype(f32) * emb_ref[2:3, :])
    fi = jnp.concatenate([x, vel, emb_class], axis=-1)
    h1 = jax.nn.relu(jnp.dot(fi, Wf1[...], preferred_element_type=f32) + bf1[...])
    gamma = jnp.dot(gsum_rep, Wg[...], preferred_element_type=f32) + bg[...]
    beta = jnp.dot(gsum_rep, Wb[...], preferred_element_type=f32) + bb[...]
    h1 = gamma * h1 + beta
    field = jnp.dot(h1, Wf2[...], preferred_element_type=f32) + bf2[...]
    eps = 1e-6
    a = vel / (jnp.sqrt(jnp.sum(vel * vel, axis=-1, keepdims=True)) + eps)
    b = field - jnp.sum(field * a, axis=-1, keepdims=True) * a
    b = b / (jnp.sqrt(jnp.sum(b * b, axis=-1, keepdims=True)) + eps)
    c = jnp.concatenate([
        a[:, 1:2] * b[:, 2:3] - a[:, 2:3] * b[:, 1:2],
        a[:, 2:3] * b[:, 0:1] - a[:, 0:1] * b[:, 2:3],
        a[:, 0:1] * b[:, 1:2] - a[:, 1:2] * b[:, 0:1],
    ], axis=-1)
    r9 = jnp.concatenate([a, b, c], axis=-1)

    def dot3(u, v):
        return jnp.sum(u * v, axis=-1, keepdims=True)

    rel_feat = jnp.concatenate([
        dot3(a, x), dot3(b, x), dot3(c, x),
        dot3(a, vel), dot3(b, vel), dot3(c, vel),
        dot3(a, field), dot3(b, field), dot3(c, field),
    ], axis=-1)
    hn = jax.nn.relu(jnp.dot(rel_feat, We[...], preferred_element_type=f32) + be[...])
    A = jnp.dot(hn, Wms[...], preferred_element_type=f32)
    B = jnp.dot(hn, Wmr[...], preferred_element_type=f32) + bm[...]
    zpad = jnp.zeros((NB, TW - 67), dtype=f32)
    stab_ref[...] = jnp.concatenate([A, x, zpad], axis=-1)
    rtab_ref[...] = jnp.concatenate([B, x, r9, jnp.zeros((NB, TW - 76), f32)], axis=-1)
    hn_ref[...] = hn
    r9_ref[...] = r9


def _edge_kernel(gs_ref, gr_ref, ea_ref, S, M1, M2, I64p, Wme4, wd, m_ref):
    f32 = jnp.float32
    gs = gs_ref[...]
    gr = gr_ref[...]
    delta = gs - gr
    t3 = jnp.dot(delta, S[...], preferred_element_type=f32)
    u0 = jnp.dot(gs + gr, I64p[...], preferred_element_type=f32)
    rlc = jnp.dot(gr * t3, M1[...], preferred_element_type=f32)
    d2 = jnp.dot(delta * t3, M2[...], preferred_element_type=f32)
    dist = jnp.sqrt(d2)
    eac = jnp.dot(ea_ref[...], Wme4[...], preferred_element_type=f32)
    m = jax.nn.relu(u0 + rlc + dist * wd[...] + eac)
    m_ref[0] = m[:, 0:32]
    m_ref[1] = m[:, 32:64]


def _final_kernel(hn_ref, agg_ref, r9_ref, x_ref, Wn, bn, Wo, bo, out_ref):
    f32 = jnp.float32
    hn = hn_ref[...]
    cat = jnp.concatenate([hn, agg_ref[0], agg_ref[1]], axis=-1)
    hn2 = hn + jax.nn.relu(jnp.dot(cat, Wn[...], preferred_element_type=f32) + bn[...])
    pred = jnp.dot(hn2, Wo[...], preferred_element_type=f32) + bo[...]
    a = r9_ref[:, 0:3]
    b = r9_ref[:, 3:6]
    c = r9_ref[:, 6:9]
    out_ref[...] = (x_ref[...] + pred[:, 0:1] * a + pred[:, 1:2] * b
                    + pred[:, 2:3] * c)


def _sc_gather(sidx, ridx, stab, rtab, gs_out, gr_out, idx_v, rows_v, sem):
    wid = lax.axis_index("s") * NC + lax.axis_index("c")

    def chunk(j):
        cb = j * GCH
        pltpu.sync_copy(sidx.at[pl.ds(cb, GCH)], idx_v)
        pltpu.async_copy(stab.at[idx_v], rows_v, sem).wait()
        pltpu.sync_copy(rows_v, gs_out.at[pl.ds(cb, GCH)])
        pltpu.sync_copy(ridx.at[pl.ds(cb, GCH)], idx_v)
        pltpu.async_copy(rtab.at[idx_v], rows_v, sem).wait()
        pltpu.sync_copy(rows_v, gr_out.at[pl.ds(cb, GCH)])

    def body(k, _):
        chunk(wid + k * NW)
        return 0

    lax.fori_loop(0, GFULL, body, 0)
    pl.when(wid < GREM)(lambda: chunk(GFULL * NW + wid))


def _sc_scatter(ridx, m_hbm, zeros_hbm, agg_out, acc, idx_v, rows_v,
                idx_t, rows_t, sem):
    c = lax.axis_index("c")
    s = lax.axis_index("s")
    rbase = s * NPT
    pltpu.sync_copy(zeros_hbm.at[pl.ds(rbase, NPT)], acc.at[pl.ds(rbase, NPT)])
    plsc.subcore_barrier()
    base = s * EPT

    def chunk(cb, iv, rv):
        pltpu.sync_copy(ridx.at[cb], iv)
        pltpu.sync_copy(m_hbm.at[c, cb], rv)
        pltpu.sync_copy(rv, acc.at[iv], add=True)

    def body(k, _):
        cb = base + k * GCH
        chunk(pl.ds(cb, GCH), idx_v, rows_v)
        return 0

    lax.fori_loop(0, SFULL, body, 0)
    chunk(pl.ds(base + SFULL * GCH, STAIL), idx_t, rows_t)
    plsc.subcore_barrier()
    pltpu.sync_copy(acc.at[pl.ds(rbase, NPT)], agg_out.at[c, pl.ds(rbase, NPT)])


def kernel(h, x, edges, vel, edge_attr_orig, charges, num_nodes, emb, Ws1,
           bs1, Ws2, bs2, Wf1, bf1, Wg, bg, Wb, bb, Wf2, bf2, We, be, Wm, bm,
           Wn, bn, Wo, bo):
    f32 = jnp.float32
    row = lambda v: v.reshape(1, -1).astype(f32)
    Wms, Wmr, Wme = Wm[0:64], Wm[64:128], Wm[128:136]

    grid1 = pl.pallas_call(
        _node_kernel,
        grid=(NBLK,),
        in_specs=[
            pl.BlockSpec((NB, D), lambda i: (i, 0)),
            pl.BlockSpec((NB, D), lambda i: (i, 0)),
            pl.BlockSpec((NB, 1), lambda i: (i, 0)),
        ] + [pl.BlockSpec(w.shape, lambda i: tuple(0 for _ in w.shape))
             for w in (emb, Ws1, row(bs1), Ws2, row(bs2), Wf1, row(bf1), Wg,
                       row(bg), Wb, row(bb), Wf2, row(bf2), We, row(be), Wms,
                       Wmr, row(bm))],
        out_specs=[
            pl.BlockSpec((NB, TW), lambda i: (i, 0)),
            pl.BlockSpec((NB, TW), lambda i: (i, 0)),
            pl.BlockSpec((NB, H), lambda i: (i, 0)),
            pl.BlockSpec((NB, 9), lambda i: (i, 0)),
        ],
        out_shape=[
            jax.ShapeDtypeStruct((N, TW), f32),
            jax.ShapeDtypeStruct((N, TW), f32),
            jax.ShapeDtypeStruct((N, H), f32),
            jax.ShapeDtypeStruct((N, 9), f32),
        ],
    )
    stab, rtab, hn, r9 = grid1(
        x.astype(f32), vel.astype(f32), charges.astype(jnp.int32), emb, Ws1,
        row(bs1), Ws2, row(bs2), Wf1, row(bf1), Wg, row(bg), Wb, row(bb), Wf2,
        row(bf2), We, row(be), Wms, Wmr, row(bm))

    sidx = edges[0]
    ridx = edges[1]

    gather = functools.partial(
        pl.kernel,
        mesh=plsc.VectorSubcoreMesh(core_axis_name="c", subcore_axis_name="s"),
        out_type=[
            jax.ShapeDtypeStruct((E, TW), f32),
            jax.ShapeDtypeStruct((E, TW), f32),
        ],
        scratch_types=[
            pltpu.VMEM((GCH,), jnp.int32),
            pltpu.VMEM((GCH, TW), f32),
            pltpu.SemaphoreType.DMA,
        ],
    )(_sc_gather)
    gs, gr = gather(sidx, ridx, stab, rtab)

    Smat = jnp.zeros((TW, TW), f32)
    for j in range(3):
        Smat = Smat.at[64 + j, 64 + j].set(1.0)
        for i in range(3):
            Smat = Smat.at[64 + j, 67 + 3 * i + j].set(1.0)
    M1 = jnp.zeros((TW, H), f32)
    for i in range(3):
        for j in range(3):
            M1 = M1.at[67 + 3 * i + j].set(Wme[i])
    M2 = jnp.zeros((TW, H), f32).at[64:67].set(1.0)
    I64p = jnp.zeros((TW, H), f32).at[jnp.arange(H), jnp.arange(H)].set(1.0)
    Wme4 = Wme[4:8]
    wd = Wme[3:4]

    grid3 = pl.pallas_call(
        _edge_kernel,
        grid=(EBLK,),
        in_specs=[
            pl.BlockSpec((EB, TW), lambda i: (i, 0)),
            pl.BlockSpec((EB, TW), lambda i: (i, 0)),
            pl.BlockSpec((EB, 4), lambda i: (i, 0)),
            pl.BlockSpec((TW, TW), lambda i: (0, 0)),
            pl.BlockSpec((TW, H), lambda i: (0, 0)),
            pl.BlockSpec((TW, H), lambda i: (0, 0)),
            pl.BlockSpec((TW, H), lambda i: (0, 0)),
            pl.BlockSpec((4, H), lambda i: (0, 0)),
            pl.BlockSpec((1, H), lambda i: (0, 0)),
        ],
        out_specs=pl.BlockSpec((2, EB, 32), lambda i: (0, i, 0)),
        out_shape=jax.ShapeDtypeStruct((2, E, 32), f32),
    )
    m2 = grid3(gs, gr, edge_attr_orig.astype(f32), Smat, M1, M2, I64p, Wme4, wd)

    scatter = functools.partial(
        pl.kernel,
        mesh=plsc.VectorSubcoreMesh(core_axis_name="c", subcore_axis_name="s"),
        compiler_params=pltpu.CompilerParams(use_tc_tiling_on_sc=False),
        out_type=jax.ShapeDtypeStruct((2, N, 32), f32),
        scratch_types=[
            pltpu.VMEM_SHARED((N, 32), f32),
            pltpu.VMEM((GCH,), jnp.int32),
            pltpu.VMEM((GCH, 32), f32),
            pltpu.VMEM((STAIL,), jnp.int32),
            pltpu.VMEM((STAIL, 32), f32),
            pltpu.SemaphoreType.DMA,
        ],
    )(_sc_scatter)
    agg2 = scatter(ridx, m2, jnp.zeros((N, 32), f32))

    grid5 = pl.pallas_call(
        _final_kernel,
        grid=(NBLK,),
        in_specs=[
            pl.BlockSpec((NB, H), lambda i: (i, 0)),
            pl.BlockSpec((2, NB, 32), lambda i: (0, i, 0)),
            pl.BlockSpec((NB, 9), lambda i: (i, 0)),
            pl.BlockSpec((NB, D), lambda i: (i, 0)),
            pl.BlockSpec((2 * H, H), lambda i: (0, 0)),
            pl.BlockSpec((1, H), lambda i: (0, 0)),
            pl.BlockSpec((H, D), lambda i: (0, 0)),
            pl.BlockSpec((1, D), lambda i: (0, 0)),
        ],
        out_specs=pl.BlockSpec((NB, D), lambda i: (i, 0)),
        out_shape=jax.ShapeDtypeStruct((N, D), f32),
    )
    return grid5(hn, agg2, r9, x.astype(f32), Wn, row(bn), Wo, row(bo))


# speedup vs baseline: 5.0712x; 1.0711x over previous
import functools

import jax
import jax.numpy as jnp
from jax import lax
from jax.experimental import pallas as pl
from jax.experimental.pallas import tpu as pltpu
from jax.experimental.pallas import tpu_sc as plsc

N = 50000
E = 800000
D = 3
H = 64
GRAPH = 100
NGB = 20
NB = NGB * GRAPH
NBLK = N // NB
EB = 8000
EBLK = E // EB

NC = 2
NS = 16
NW = NC * NS
GCH = 128
NCHUNK = E // GCH
GFULL = NCHUNK // NW
GREM = NCHUNK - GFULL * NW
SFULL = NCHUNK // NS
SREM = NCHUNK - SFULL * NS
EPT = E // NS
SFULL2 = EPT // GCH
STAIL = EPT - SFULL2 * GCH
NPT = N // NS

TW = 128


def _node_kernel(x_ref, vel_ref, ch_ref, emb_ref, Ws1, bs1, Ws2, bs2,
                 Wf1, bf1, Wg, bg, Wb, bb, Wf2, bf2, We, be,
                 Wms, Wmr, bm, stab_ref, rtab_ref, hn_ref, r9_ref):
    f32 = jnp.float32
    x = x_ref[...]
    vel = vel_ref[...]
    inputs = jnp.concatenate([x, vel], axis=-1)
    hs = jnp.tanh(jnp.dot(inputs, Ws1[...], preferred_element_type=f32) + bs1[...])
    hs = jnp.dot(hs, Ws2[...], preferred_element_type=f32) + bs2[...]
    row_g = lax.broadcasted_iota(jnp.int32, (NGB, NB), 0)
    col_g = lax.broadcasted_iota(jnp.int32, (NGB, NB), 1) // GRAPH
    P = jnp.where(row_g == col_g, 1.0 / GRAPH, 0.0).astype(f32)
    gsum = jnp.dot(P, hs, preferred_element_type=f32)
    row_n = lax.broadcasted_iota(jnp.int32, (NB, NGB), 0) // GRAPH
    col_n = lax.broadcasted_iota(jnp.int32, (NB, NGB), 1)
    Q = jnp.where(row_n == col_n, 1.0, 0.0).astype(f32)
    gsum_rep = jnp.dot(Q, gsum, preferred_element_type=f32)
    idx = ch_ref[...] + 1
    emb_class = ((idx == 0).astype(f32) * emb_ref[0:1, :]
                 + (idx == 1).astype(f32) * emb_ref[1:2, :]
                 + (idx == 2).astype(f32) * emb_ref[2:3, :])
    fi = jnp.concatenate([x, vel, emb_class], axis=-1)
    h1 = jax.nn.relu(jnp.dot(fi, Wf1[...], preferred_element_type=f32) + bf1[...])
    gamma = jnp.dot(gsum_rep, Wg[...], preferred_element_type=f32) + bg[...]
    beta = jnp.dot(gsum_rep, Wb[...], preferred_element_type=f32) + bb[...]
    h1 = gamma * h1 + beta
    field = jnp.dot(h1, Wf2[...], preferred_element_type=f32) + bf2[...]
    eps = 1e-6
    a = vel / (jnp.sqrt(jnp.sum(vel * vel, axis=-1, keepdims=True)) + eps)
    b = field - jnp.sum(field * a, axis=-1, keepdims=True) * a
    b = b / (jnp.sqrt(jnp.sum(b * b, axis=-1, keepdims=True)) + eps)
    c = jnp.concatenate([
        a[:, 1:2] * b[:, 2:3] - a[:, 2:3] * b[:, 1:2],
        a[:, 2:3] * b[:, 0:1] - a[:, 0:1] * b[:, 2:3],
        a[:, 0:1] * b[:, 1:2] - a[:, 1:2] * b[:, 0:1],
    ], axis=-1)
    r9 = jnp.concatenate([a, b, c], axis=-1)

    def dot3(u, v):
        return jnp.sum(u * v, axis=-1, keepdims=True)

    rel_feat = jnp.concatenate([
        dot3(a, x), dot3(b, x), dot3(c, x),
        dot3(a, vel), dot3(b, vel), dot3(c, vel),
        dot3(a, field), dot3(b, field), dot3(c, field),
    ], axis=-1)
    hn = jax.nn.relu(jnp.dot(rel_feat, We[...], preferred_element_type=f32) + be[...])
    A = jnp.dot(hn, Wms[...], preferred_element_type=f32)
    B = jnp.dot(hn, Wmr[...], preferred_element_type=f32) + bm[...]
    zpad = jnp.zeros((NB, TW - 67), dtype=f32)
    stab_ref[...] = jnp.concatenate([A, x, zpad], axis=-1)
    rtab_ref[...] = jnp.concatenate([B, x, r9, jnp.zeros((NB, TW - 76), f32)], axis=-1)
    hn_ref[...] = hn
    r9_ref[...] = r9


def _edge_kernel(gs_ref, gr_ref, ea_ref, S, M1, M2, I64p, Wme4, wd, m_ref):
    f32 = jnp.float32
    gs = gs_ref[...]
    gr = gr_ref[...]
    delta = gs - gr
    t3 = jnp.dot(delta, S[...], preferred_element_type=f32)
    u0 = jnp.dot(gs + gr, I64p[...], preferred_element_type=f32)
    rlc = jnp.dot(gr * t3, M1[...], preferred_element_type=f32)
    d2 = jnp.dot(delta * t3, M2[...], preferred_element_type=f32)
    dist = jnp.sqrt(d2)
    eac = jnp.dot(ea_ref[...], Wme4[...], preferred_element_type=f32)
    m = jax.nn.relu(u0 + rlc + dist * wd[...] + eac)
    m_ref[0] = m[:, 0:32]
    m_ref[1] = m[:, 32:64]


def _final_kernel(hn_ref, agg_ref, r9_ref, x_ref, Wn, bn, Wo, bo, out_ref):
    f32 = jnp.float32
    hn = hn_ref[...]
    cat = jnp.concatenate([hn, agg_ref[0], agg_ref[1]], axis=-1)
    hn2 = hn + jax.nn.relu(jnp.dot(cat, Wn[...], preferred_element_type=f32) + bn[...])
    pred = jnp.dot(hn2, Wo[...], preferred_element_type=f32) + bo[...]
    a = r9_ref[:, 0:3]
    b = r9_ref[:, 3:6]
    c = r9_ref[:, 6:9]
    out_ref[...] = (x_ref[...] + pred[:, 0:1] * a + pred[:, 1:2] * b
                    + pred[:, 2:3] * c)


def _sc_gather(sidx, ridx, stab, rtab, gs_out, gr_out, idx_s, idx_r,
               rows_s, rows_r, gsem1, gsem2, wsem1, wsem2):
    wid = lax.axis_index("s") * NC + lax.axis_index("c")

    def chunk(j):
        cb = j * GCH
        pltpu.sync_copy(sidx.at[pl.ds(cb, GCH)], idx_s)
        pltpu.sync_copy(ridx.at[pl.ds(cb, GCH)], idx_r)
        h1 = pltpu.async_copy(stab.at[idx_s], rows_s, gsem1)
        h2 = pltpu.async_copy(rtab.at[idx_r], rows_r, gsem2)
        h1.wait()
        h3 = pltpu.async_copy(rows_s, gs_out.at[pl.ds(cb, GCH)], wsem1)
        h2.wait()
        h4 = pltpu.async_copy(rows_r, gr_out.at[pl.ds(cb, GCH)], wsem2)
        h3.wait()
        h4.wait()

    def body(k, _):
        chunk(wid + k * NW)
        return 0

    lax.fori_loop(0, GFULL, body, 0)
    pl.when(wid < GREM)(lambda: chunk(GFULL * NW + wid))


def _sc_scatter(ridx, m_hbm, zeros_hbm, agg_out, acc, idx_v, rows_v,
                idx_t, rows_t, sem):
    c = lax.axis_index("c")
    s = lax.axis_index("s")
    rbase = s * NPT
    pltpu.sync_copy(zeros_hbm.at[pl.ds(rbase, NPT)], acc.at[pl.ds(rbase, NPT)])
    plsc.subcore_barrier()
    base = s * EPT

    def chunk(cb, iv, rv):
        pltpu.sync_copy(ridx.at[cb], iv)
        pltpu.sync_copy(m_hbm.at[c, cb], rv)
        pltpu.sync_copy(rv, acc.at[iv], add=True)

    def body(k, _):
        cb = base + k * GCH
        chunk(pl.ds(cb, GCH), idx_v, rows_v)
        return 0

    lax.fori_loop(0, SFULL, body, 0)
    chunk(pl.ds(base + SFULL * GCH, STAIL), idx_t, rows_t)
    plsc.subcore_barrier()
    pltpu.sync_copy(acc.at[pl.ds(rbase, NPT)], agg_out.at[c, pl.ds(rbase, NPT)])


def kernel(h, x, edges, vel, edge_attr_orig, charges, num_nodes, emb, Ws1,
           bs1, Ws2, bs2, Wf1, bf1, Wg, bg, Wb, bb, Wf2, bf2, We, be, Wm, bm,
           Wn, bn, Wo, bo):
    f32 = jnp.float32
    row = lambda v: v.reshape(1, -1).astype(f32)
    Wms, Wmr, Wme = Wm[0:64], Wm[64:128], Wm[128:136]

    grid1 = pl.pallas_call(
        _node_kernel,
        grid=(NBLK,),
        in_specs=[
            pl.BlockSpec((NB, D), lambda i: (i, 0)),
            pl.BlockSpec((NB, D), lambda i: (i, 0)),
            pl.BlockSpec((NB, 1), lambda i: (i, 0)),
        ] + [pl.BlockSpec(w.shape, lambda i: tuple(0 for _ in w.shape))
             for w in (emb, Ws1, row(bs1), Ws2, row(bs2), Wf1, row(bf1), Wg,
                       row(bg), Wb, row(bb), Wf2, row(bf2), We, row(be), Wms,
                       Wmr, row(bm))],
        out_specs=[
            pl.BlockSpec((NB, TW), lambda i: (i, 0)),
            pl.BlockSpec((NB, TW), lambda i: (i, 0)),
            pl.BlockSpec((NB, H), lambda i: (i, 0)),
            pl.BlockSpec((NB, 9), lambda i: (i, 0)),
        ],
        out_shape=[
            jax.ShapeDtypeStruct((N, TW), f32),
            jax.ShapeDtypeStruct((N, TW), f32),
            jax.ShapeDtypeStruct((N, H), f32),
            jax.ShapeDtypeStruct((N, 9), f32),
        ],
    )
    stab, rtab, hn, r9 = grid1(
        x.astype(f32), vel.astype(f32), charges.astype(jnp.int32), emb, Ws1,
        row(bs1), Ws2, row(bs2), Wf1, row(bf1), Wg, row(bg), Wb, row(bb), Wf2,
        row(bf2), We, row(be), Wms, Wmr, row(bm))

    sidx = edges[0]
    ridx = edges[1]

    gather = functools.partial(
        pl.kernel,
        mesh=plsc.VectorSubcoreMesh(core_axis_name="c", subcore_axis_name="s"),
        out_type=[
            jax.ShapeDtypeStruct((E, TW), f32),
            jax.ShapeDtypeStruct((E, TW), f32),
        ],
        scratch_types=[
            pltpu.VMEM((GCH,), jnp.int32),
            pltpu.VMEM((GCH,), jnp.int32),
            pltpu.VMEM((GCH, TW), f32),
            pltpu.VMEM((GCH, TW), f32),
            pltpu.SemaphoreType.DMA,
            pltpu.SemaphoreType.DMA,
            pltpu.SemaphoreType.DMA,
            pltpu.SemaphoreType.DMA,
        ],
    )(_sc_gather)
    gs, gr = gather(sidx, ridx, stab, rtab)

    Smat = jnp.zeros((TW, TW), f32)
    for j in range(3):
        Smat = Smat.at[64 + j, 64 + j].set(1.0)
        for i in range(3):
            Smat = Smat.at[64 + j, 67 + 3 * i + j].set(1.0)
    M1 = jnp.zeros((TW, H), f32)
    for i in range(3):
        for j in range(3):
            M1 = M1.at[67 + 3 * i + j].set(Wme[i])
    M2 = jnp.zeros((TW, H), f32).at[64:67].set(1.0)
    I64p = jnp.zeros((TW, H), f32).at[jnp.arange(H), jnp.arange(H)].set(1.0)
    Wme4 = Wme[4:8]
    wd = Wme[3:4]

    grid3 = pl.pallas_call(
        _edge_kernel,
        grid=(EBLK,),
        in_specs=[
            pl.BlockSpec((EB, TW), lambda i: (i, 0)),
            pl.BlockSpec((EB, TW), lambda i: (i, 0)),
            pl.BlockSpec((EB, 4), lambda i: (i, 0)),
            pl.BlockSpec((TW, TW), lambda i: (0, 0)),
            pl.BlockSpec((TW, H), lambda i: (0, 0)),
            pl.BlockSpec((TW, H), lambda i: (0, 0)),
            pl.BlockSpec((TW, H), lambda i: (0, 0)),
            pl.BlockSpec((4, H), lambda i: (0, 0)),
            pl.BlockSpec((1, H), lambda i: (0, 0)),
        ],
        out_specs=pl.BlockSpec((2, EB, 32), lambda i: (0, i, 0)),
        out_shape=jax.ShapeDtypeStruct((2, E, 32), f32),
    )
    m2 = grid3(gs, gr, edge_attr_orig.astype(f32), Smat, M1, M2, I64p, Wme4, wd)

    scatter = functools.partial(
        pl.kernel,
        mesh=plsc.VectorSubcoreMesh(core_axis_name="c", subcore_axis_name="s"),
        compiler_params=pltpu.CompilerParams(use_tc_tiling_on_sc=False),
        out_type=jax.ShapeDtypeStruct((2, N, 32), f32),
        scratch_types=[
            pltpu.VMEM_SHARED((N, 32), f32),
            pltpu.VMEM((GCH,), jnp.int32),
            pltpu.VMEM((GCH, 32), f32),
            pltpu.VMEM((STAIL,), jnp.int32),
            pltpu.VMEM((STAIL, 32), f32),
            pltpu.SemaphoreType.DMA,
        ],
    )(_sc_scatter)
    agg2 = scatter(ridx, m2, jnp.zeros((N, 32), f32))

    grid5 = pl.pallas_call(
        _final_kernel,
        grid=(NBLK,),
        in_specs=[
            pl.BlockSpec((NB, H), lambda i: (i, 0)),
            pl.BlockSpec((2, NB, 32), lambda i: (0, i, 0)),
            pl.BlockSpec((NB, 9), lambda i: (i, 0)),
            pl.BlockSpec((NB, D), lambda i: (i, 0)),
            pl.BlockSpec((2 * H, H), lambda i: (0, 0)),
            pl.BlockSpec((1, H), lambda i: (0, 0)),
            pl.BlockSpec((H, D), lambda i: (0, 0)),
            pl.BlockSpec((1, D), lambda i: (0, 0)),
        ],
        out_specs=pl.BlockSpec((NB, D), lambda i: (i, 0)),
        out_shape=jax.ShapeDtypeStruct((N, D), f32),
    )
    return grid5(hn, agg2, r9, x.astype(f32), Wn, row(bn), Wo, row(bo))


# speedup vs baseline: 5.4017x; 1.0652x over previous
import functools

import jax
import jax.numpy as jnp
from jax import lax
from jax.experimental import pallas as pl
from jax.experimental.pallas import tpu as pltpu
from jax.experimental.pallas import tpu_sc as plsc

N = 50000
E = 800000
D = 3
H = 64
GRAPH = 100
NGB = 20
NB = NGB * GRAPH
NBLK = N // NB
EB = 8000
EBLK = E // EB

NC = 2
NS = 16
NW = NC * NS
GCH = 128
NCHUNK = E // GCH
GFULL = NCHUNK // NW
GREM = NCHUNK - GFULL * NW
SFULL = NCHUNK // NS
SREM = NCHUNK - SFULL * NS
EPT = E // NS
SFULL2 = EPT // GCH
STAIL = EPT - SFULL2 * GCH
NPT = N // NS

TW = 128


def _node_kernel(x_ref, vel_ref, ch_ref, emb_ref, Ws1, bs1, Ws2, bs2,
                 Wf1, bf1, Wg, bg, Wb, bb, Wf2, bf2, We, be,
                 Wms, Wmr, bm, stab_ref, rtab_ref, hn_ref, r9_ref):
    f32 = jnp.float32
    x = x_ref[...]
    vel = vel_ref[...]
    inputs = jnp.concatenate([x, vel], axis=-1)
    hs = jnp.tanh(jnp.dot(inputs, Ws1[...], preferred_element_type=f32) + bs1[...])
    hs = jnp.dot(hs, Ws2[...], preferred_element_type=f32) + bs2[...]
    row_g = lax.broadcasted_iota(jnp.int32, (NGB, NB), 0)
    col_g = lax.broadcasted_iota(jnp.int32, (NGB, NB), 1) // GRAPH
    P = jnp.where(row_g == col_g, 1.0 / GRAPH, 0.0).astype(f32)
    gsum = jnp.dot(P, hs, preferred_element_type=f32)
    row_n = lax.broadcasted_iota(jnp.int32, (NB, NGB), 0) // GRAPH
    col_n = lax.broadcasted_iota(jnp.int32, (NB, NGB), 1)
    Q = jnp.where(row_n == col_n, 1.0, 0.0).astype(f32)
    gsum_rep = jnp.dot(Q, gsum, preferred_element_type=f32)
    idx = ch_ref[...] + 1
    emb_class = ((idx == 0).astype(f32) * emb_ref[0:1, :]
                 + (idx == 1).astype(f32) * emb_ref[1:2, :]
                 + (idx == 2).astype(f32) * emb_ref[2:3, :])
    fi = jnp.concatenate([x, vel, emb_class], axis=-1)
    h1 = jax.nn.relu(jnp.dot(fi, Wf1[...], preferred_element_type=f32) + bf1[...])
    gamma = jnp.dot(gsum_rep, Wg[...], preferred_element_type=f32) + bg[...]
    beta = jnp.dot(gsum_rep, Wb[...], preferred_element_type=f32) + bb[...]
    h1 = gamma * h1 + beta
    field = jnp.dot(h1, Wf2[...], preferred_element_type=f32) + bf2[...]
    eps = 1e-6
    a = vel / (jnp.sqrt(jnp.sum(vel * vel, axis=-1, keepdims=True)) + eps)
    b = field - jnp.sum(field * a, axis=-1, keepdims=True) * a
    b = b / (jnp.sqrt(jnp.sum(b * b, axis=-1, keepdims=True)) + eps)
    c = jnp.concatenate([
        a[:, 1:2] * b[:, 2:3] - a[:, 2:3] * b[:, 1:2],
        a[:, 2:3] * b[:, 0:1] - a[:, 0:1] * b[:, 2:3],
        a[:, 0:1] * b[:, 1:2] - a[:, 1:2] * b[:, 0:1],
    ], axis=-1)
    r9 = jnp.concatenate([a, b, c], axis=-1)

    def dot3(u, v):
        return jnp.sum(u * v, axis=-1, keepdims=True)

    rel_feat = jnp.concatenate([
        dot3(a, x), dot3(b, x), dot3(c, x),
        dot3(a, vel), dot3(b, vel), dot3(c, vel),
        dot3(a, field), dot3(b, field), dot3(c, field),
    ], axis=-1)
    hn = jax.nn.relu(jnp.dot(rel_feat, We[...], preferred_element_type=f32) + be[...])
    A = jnp.dot(hn, Wms[...], preferred_element_type=f32)
    B = jnp.dot(hn, Wmr[...], preferred_element_type=f32) + bm[...]
    zpad = jnp.zeros((NB, TW - 67), dtype=f32)
    stab_ref[...] = jnp.concatenate([A, x, zpad], axis=-1)
    rtab_ref[...] = jnp.concatenate([B, x, r9, jnp.zeros((NB, TW - 76), f32)], axis=-1)
    hn_ref[...] = hn
    r9_ref[...] = r9


def _edge_kernel(gs_ref, gr_ref, ea_ref, S, M1, M2, I64p, Wme4, wd, m_ref):
    f32 = jnp.float32
    gs = gs_ref[...]
    gr = gr_ref[...]
    delta = gs - gr
    t3 = jnp.dot(delta, S[...], preferred_element_type=f32)
    u0 = jnp.dot(gs + gr, I64p[...], preferred_element_type=f32)
    rlc = jnp.dot(gr * t3, M1[...], preferred_element_type=f32)
    d2 = jnp.dot(delta * t3, M2[...], preferred_element_type=f32)
    dist = jnp.sqrt(d2)
    eac = jnp.dot(ea_ref[...], Wme4[...], preferred_element_type=f32)
    m = jax.nn.relu(u0 + rlc + dist * wd[...] + eac)
    m_ref[0] = m[:, 0:32]
    m_ref[1] = m[:, 32:64]


def _final_kernel(hn_ref, agg_ref, r9_ref, x_ref, Wn, bn, Wo, bo, out_ref):
    f32 = jnp.float32
    hn = hn_ref[...]
    cat = jnp.concatenate([hn, agg_ref[0], agg_ref[1]], axis=-1)
    hn2 = hn + jax.nn.relu(jnp.dot(cat, Wn[...], preferred_element_type=f32) + bn[...])
    pred = jnp.dot(hn2, Wo[...], preferred_element_type=f32) + bo[...]
    a = r9_ref[:, 0:3]
    b = r9_ref[:, 3:6]
    c = r9_ref[:, 6:9]
    out_ref[...] = (x_ref[...] + pred[:, 0:1] * a + pred[:, 1:2] * b
                    + pred[:, 2:3] * c)


def _sc_gather(sidx, ridx, stab, rtab, gs_out, gr_out, idx_s, idx_r,
               rows_s, rows_r, gsem1, gsem2, wsem1, wsem2):
    wid = lax.axis_index("s") * NC + lax.axis_index("c")

    def chunk(j):
        cb = j * GCH
        pltpu.sync_copy(sidx.at[pl.ds(cb, GCH)], idx_s)
        pltpu.sync_copy(ridx.at[pl.ds(cb, GCH)], idx_r)
        h1 = pltpu.async_copy(stab.at[idx_s], rows_s, gsem1)
        h2 = pltpu.async_copy(rtab.at[idx_r], rows_r, gsem2)
        h1.wait()
        h3 = pltpu.async_copy(rows_s, gs_out.at[pl.ds(cb, GCH)], wsem1)
        h2.wait()
        h4 = pltpu.async_copy(rows_r, gr_out.at[pl.ds(cb, GCH)], wsem2)
        h3.wait()
        h4.wait()

    def body(k, _):
        chunk(wid + k * NW)
        return 0

    lax.fori_loop(0, GFULL, body, 0)
    pl.when(wid < GREM)(lambda: chunk(GFULL * NW + wid))


def _sc_scatter(ridx, m_hbm, zeros_hbm, agg_out, acc, idx_0, idx_1, rows_0,
                rows_1, idx_t, rows_t, msem1, msem2, asem1, asem2):
    c = lax.axis_index("c")
    s = lax.axis_index("s")
    rbase = s * NPT
    pltpu.sync_copy(zeros_hbm.at[pl.ds(rbase, NPT)], acc.at[pl.ds(rbase, NPT)])
    plsc.subcore_barrier()
    base = s * EPT

    def body(k2, _):
        cb0 = pl.ds(base + (2 * k2) * GCH, GCH)
        cb1 = pl.ds(base + (2 * k2 + 1) * GCH, GCH)
        pltpu.sync_copy(ridx.at[cb0], idx_0)
        h0 = pltpu.async_copy(m_hbm.at[c, cb0], rows_0, msem1)
        pltpu.sync_copy(ridx.at[cb1], idx_1)
        h1 = pltpu.async_copy(m_hbm.at[c, cb1], rows_1, msem2)
        h0.wait()
        a0 = pltpu.async_copy(rows_0, acc.at[idx_0], asem1, add=True)
        h1.wait()
        a1 = pltpu.async_copy(rows_1, acc.at[idx_1], asem2, add=True)
        a0.wait()
        a1.wait()
        return 0

    lax.fori_loop(0, SFULL // 2, body, 0)
    cbt = pl.ds(base + SFULL * GCH, STAIL)
    pltpu.sync_copy(ridx.at[cbt], idx_t)
    pltpu.sync_copy(m_hbm.at[c, cbt], rows_t)
    pltpu.sync_copy(rows_t, acc.at[idx_t], add=True)
    plsc.subcore_barrier()
    pltpu.sync_copy(acc.at[pl.ds(rbase, NPT)], agg_out.at[c, pl.ds(rbase, NPT)])


def kernel(h, x, edges, vel, edge_attr_orig, charges, num_nodes, emb, Ws1,
           bs1, Ws2, bs2, Wf1, bf1, Wg, bg, Wb, bb, Wf2, bf2, We, be, Wm, bm,
           Wn, bn, Wo, bo):
    f32 = jnp.float32
    row = lambda v: v.reshape(1, -1).astype(f32)
    Wms, Wmr, Wme = Wm[0:64], Wm[64:128], Wm[128:136]

    grid1 = pl.pallas_call(
        _node_kernel,
        grid=(NBLK,),
        in_specs=[
            pl.BlockSpec((NB, D), lambda i: (i, 0)),
            pl.BlockSpec((NB, D), lambda i: (i, 0)),
            pl.BlockSpec((NB, 1), lambda i: (i, 0)),
        ] + [pl.BlockSpec(w.shape, lambda i: tuple(0 for _ in w.shape))
             for w in (emb, Ws1, row(bs1), Ws2, row(bs2), Wf1, row(bf1), Wg,
                       row(bg), Wb, row(bb), Wf2, row(bf2), We, row(be), Wms,
                       Wmr, row(bm))],
        out_specs=[
            pl.BlockSpec((NB, TW), lambda i: (i, 0)),
            pl.BlockSpec((NB, TW), lambda i: (i, 0)),
            pl.BlockSpec((NB, H), lambda i: (i, 0)),
            pl.BlockSpec((NB, 9), lambda i: (i, 0)),
        ],
        out_shape=[
            jax.ShapeDtypeStruct((N, TW), f32),
            jax.ShapeDtypeStruct((N, TW), f32),
            jax.ShapeDtypeStruct((N, H), f32),
            jax.ShapeDtypeStruct((N, 9), f32),
        ],
    )
    stab, rtab, hn, r9 = grid1(
        x.astype(f32), vel.astype(f32), charges.astype(jnp.int32), emb, Ws1,
        row(bs1), Ws2, row(bs2), Wf1, row(bf1), Wg, row(bg), Wb, row(bb), Wf2,
        row(bf2), We, row(be), Wms, Wmr, row(bm))

    sidx = edges[0]
    ridx = edges[1]

    gather = functools.partial(
        pl.kernel,
        mesh=plsc.VectorSubcoreMesh(core_axis_name="c", subcore_axis_name="s"),
        out_type=[
            jax.ShapeDtypeStruct((E, TW), f32),
            jax.ShapeDtypeStruct((E, TW), f32),
        ],
        scratch_types=[
            pltpu.VMEM((GCH,), jnp.int32),
            pltpu.VMEM((GCH,), jnp.int32),
            pltpu.VMEM((GCH, TW), f32),
            pltpu.VMEM((GCH, TW), f32),
            pltpu.SemaphoreType.DMA,
            pltpu.SemaphoreType.DMA,
            pltpu.SemaphoreType.DMA,
            pltpu.SemaphoreType.DMA,
        ],
    )(_sc_gather)
    gs, gr = gather(sidx, ridx, stab, rtab)

    Smat = jnp.zeros((TW, TW), f32)
    for j in range(3):
        Smat = Smat.at[64 + j, 64 + j].set(1.0)
        for i in range(3):
            Smat = Smat.at[64 + j, 67 + 3 * i + j].set(1.0)
    M1 = jnp.zeros((TW, H), f32)
    for i in range(3):
        for j in range(3):
            M1 = M1.at[67 + 3 * i + j].set(Wme[i])
    M2 = jnp.zeros((TW, H), f32).at[64:67].set(1.0)
    I64p = jnp.zeros((TW, H), f32).at[jnp.arange(H), jnp.arange(H)].set(1.0)
    Wme4 = Wme[4:8]
    wd = Wme[3:4]

    grid3 = pl.pallas_call(
        _edge_kernel,
        grid=(EBLK,),
        in_specs=[
            pl.BlockSpec((EB, TW), lambda i: (i, 0)),
            pl.BlockSpec((EB, TW), lambda i: (i, 0)),
            pl.BlockSpec((EB, 4), lambda i: (i, 0)),
            pl.BlockSpec((TW, TW), lambda i: (0, 0)),
            pl.BlockSpec((TW, H), lambda i: (0, 0)),
            pl.BlockSpec((TW, H), lambda i: (0, 0)),
            pl.BlockSpec((TW, H), lambda i: (0, 0)),
            pl.BlockSpec((4, H), lambda i: (0, 0)),
            pl.BlockSpec((1, H), lambda i: (0, 0)),
        ],
        out_specs=pl.BlockSpec((2, EB, 32), lambda i: (0, i, 0)),
        out_shape=jax.ShapeDtypeStruct((2, E, 32), f32),
    )
    m2 = grid3(gs, gr, edge_attr_orig.astype(f32), Smat, M1, M2, I64p, Wme4, wd)

    scatter = functools.partial(
        pl.kernel,
        mesh=plsc.VectorSubcoreMesh(core_axis_name="c", subcore_axis_name="s"),
        compiler_params=pltpu.CompilerParams(use_tc_tiling_on_sc=False),
        out_type=jax.ShapeDtypeStruct((2, N, 32), f32),
        scratch_types=[
            pltpu.VMEM_SHARED((N, 32), f32),
            pltpu.VMEM((GCH,), jnp.int32),
            pltpu.VMEM((GCH,), jnp.int32),
            pltpu.VMEM((GCH, 32), f32),
            pltpu.VMEM((GCH, 32), f32),
            pltpu.VMEM((STAIL,), jnp.int32),
            pltpu.VMEM((STAIL, 32), f32),
            pltpu.SemaphoreType.DMA,
            pltpu.SemaphoreType.DMA,
            pltpu.SemaphoreType.DMA,
            pltpu.SemaphoreType.DMA,
        ],
    )(_sc_scatter)
    agg2 = scatter(ridx, m2, jnp.zeros((N, 32), f32))

    grid5 = pl.pallas_call(
        _final_kernel,
        grid=(NBLK,),
        in_specs=[
            pl.BlockSpec((NB, H), lambda i: (i, 0)),
            pl.BlockSpec((2, NB, 32), lambda i: (0, i, 0)),
            pl.BlockSpec((NB, 9), lambda i: (i, 0)),
            pl.BlockSpec((NB, D), lambda i: (i, 0)),
            pl.BlockSpec((2 * H, H), lambda i: (0, 0)),
            pl.BlockSpec((1, H), lambda i: (0, 0)),
            pl.BlockSpec((H, D), lambda i: (0, 0)),
            pl.BlockSpec((1, D), lambda i: (0, 0)),
        ],
        out_specs=pl.BlockSpec((NB, D), lambda i: (i, 0)),
        out_shape=jax.ShapeDtypeStruct((N, D), f32),
    )
    return grid5(hn, agg2, r9, x.astype(f32), Wn, row(bn), Wo, row(bo))
